# baseline (speedup 1.0000x reference)
import functools
import math

import jax
import jax.numpy as jnp
from jax import lax
from jax.experimental import pallas as pl
from jax.experimental.pallas import tpu as pltpu
from jax.experimental.pallas import tpu_sc as plsc

N_NODES = 50000
N_EDGES = 800000
R = 4
C = 8
TEMPERATURE = 2.0 / 3.0
LIMIT_A = -0.1
LIMIT_B = 1.1

E_PAD = 819200
N_WORKERS = 32
CHUNK = 3200
CHUNKS_PER_W = E_PAD // (N_WORKERS * CHUNK)
ROWS128 = E_PAD // 128
ROWS_OUT = N_EDGES // 128
NPACK = N_NODES // 16

_THRESHOLD = TEMPERATURE * math.log(-LIMIT_A / LIMIT_B)


def _tc_pre_body(icl_ref, la_ref, ap_ref, dst_ref, typ_ref,
                 z_ref, l0_ref, p_ref, fused_ref):
    @pl.when(pl.program_id(0) == 0)
    def _():
        la = la_ref[...]
        z = jnp.clip(jax.nn.sigmoid(la) * (LIMIT_B - LIMIT_A) + LIMIT_A,
                     0.0, 1.0)
        w = jax.nn.sigmoid(icl_ref[...]) * z
        z_ref[...] = z
        l0_ref[...] = jax.nn.sigmoid(la - _THRESHOLD)

        e1 = (lax.broadcasted_iota(jnp.int32, (128, C), 0) % C
              == lax.broadcasted_iota(jnp.int32, (128, C), 1)
              ).astype(jnp.float32)
        e2 = (lax.broadcasted_iota(jnp.int32, (C, 128), 0)
              == lax.broadcasted_iota(jnp.int32, (C, 128), 1) % C
              ).astype(jnp.float32)
        blk = (lax.broadcasted_iota(jnp.int32, (128, 128), 0) // C
               == lax.broadcasted_iota(jnp.int32, (128, 128), 1) // C
               ).astype(jnp.float32)

        ap = ap_ref[...]
        for t in range(R):
            wt_tiled = jnp.dot(jnp.dot(e1, w[t].T,
                                       preferred_element_type=jnp.float32),
                               e2, preferred_element_type=jnp.float32)
            bd = wt_tiled * blk
            p_ref[t] = jnp.dot(ap, bd, preferred_element_type=jnp.float32)

    fused_ref[...] = dst_ref[...] + N_NODES * typ_ref[...]


def _tc_pre(icl, la, a_packed, dst2d, typ2d):
    grid = 5
    be = ROWS128 // grid
    return pl.pallas_call(
        _tc_pre_body,
        grid=(grid,),
        in_specs=[
            pl.BlockSpec((R, C, C), lambda i: (0, 0, 0)),
            pl.BlockSpec((R, C, C), lambda i: (0, 0, 0)),
            pl.BlockSpec((NPACK, 128), lambda i: (0, 0)),
            pl.BlockSpec((be, 128), lambda i: (i, 0)),
            pl.BlockSpec((be, 128), lambda i: (i, 0)),
        ],
        out_specs=[
            pl.BlockSpec((R, C, C), lambda i: (0, 0, 0)),
            pl.BlockSpec((R, C, C), lambda i: (0, 0, 0)),
            pl.BlockSpec((R, NPACK, 128), lambda i: (0, 0, 0)),
            pl.BlockSpec((be, 128), lambda i: (i, 0)),
        ],
        out_shape=[
            jax.ShapeDtypeStruct((R, C, C), jnp.float32),
            jax.ShapeDtypeStruct((R, C, C), jnp.float32),
            jax.ShapeDtypeStruct((R, NPACK, 128), jnp.float32),
            jax.ShapeDtypeStruct((ROWS128, 128), jnp.int32),
        ],
    )(icl, la, a_packed, dst2d, typ2d)


_J = CHUNK // 128


def _sc_body(a_hbm, p_hbm, sidx_hbm, fidx_hbm, out_hbm,
             sidx_v, fidx_v, srows_v, drows_v, lg_v, sems):
    wid = lax.axis_index("s") * 2 + lax.axis_index("c")

    def start_chunk(g, b):
        row0 = wid * (CHUNKS_PER_W * _J) + g * _J
        pltpu.sync_copy(sidx_hbm.at[pl.ds(row0, _J)], sidx_v.at[b])
        pltpu.sync_copy(fidx_hbm.at[pl.ds(row0, _J)], fidx_v.at[b])
        for j in range(_J):
            pltpu.async_copy(a_hbm.at[sidx_v.at[b].at[j]],
                             srows_v.at[b].at[pl.ds(j * 128, 128)], sems.at[b])
            pltpu.async_copy(p_hbm.at[fidx_v.at[b].at[j]],
                             drows_v.at[b].at[pl.ds(j * 128, 128)], sems.at[b])

    def finish_chunk(g, b):
        pltpu.make_async_copy(
            a_hbm.at[pl.ds(0, CHUNK)],
            srows_v.at[b].at[pl.ds(0, CHUNK)], sems.at[b]).wait()
        pltpu.make_async_copy(
            p_hbm.at[pl.ds(0, CHUNK)],
            drows_v.at[b].at[pl.ds(0, CHUNK)], sems.at[b]).wait()

        def vec_body(v, _):
            ids = lax.iota(jnp.int32, 16) + v * 16
            acc = jnp.zeros((16,), jnp.float32)
            for c in range(C):
                cc = jnp.full((16,), c, jnp.int32)
                s = plsc.load_gather(srows_v.at[b], [ids, cc])
                d = plsc.load_gather(drows_v.at[b], [ids, cc])
                acc = acc + s * d
            lg_v[pl.ds(v * 16, 16)] = acc
            return 0

        lax.fori_loop(0, CHUNK // 16, vec_body, 0, unroll=4)
        base = wid * (CHUNKS_PER_W * CHUNK) + g * CHUNK
        pltpu.sync_copy(lg_v, out_hbm.at[pl.ds(base, CHUNK)])

    start_chunk(0, 0)

    def pair_body(h, _):
        g = h * 2
        start_chunk(g + 1, 1)
        finish_chunk(g, 0)
        start_chunk(g + 2, 0)
        finish_chunk(g + 1, 1)
        return 0

    n_pairs = CHUNKS_PER_W // 2 - 1
    lax.fori_loop(0, n_pairs, pair_body, 0, unroll=False)
    g0 = 2 * n_pairs
    start_chunk(g0 + 1, 1)
    finish_chunk(g0, 0)
    if CHUNKS_PER_W % 2:
        start_chunk(g0 + 2, 0)
        finish_chunk(g0 + 1, 1)
        finish_chunk(g0 + 2, 0)
    else:
        finish_chunk(g0 + 1, 1)


def _sc_main(a_flat, p_flat, sidx2d, fidx2d):
    mesh = plsc.VectorSubcoreMesh(core_axis_name="c", subcore_axis_name="s")
    kfn = functools.partial(
        pl.kernel,
        out_type=jax.ShapeDtypeStruct((E_PAD,), jnp.float32),
        mesh=mesh,
        compiler_params=pltpu.CompilerParams(
            use_tc_tiling_on_sc=False, needs_layout_passes=False),
        scratch_types=[
            pltpu.VMEM((2, _J, 128), jnp.int32),
            pltpu.VMEM((2, _J, 128), jnp.int32),
            pltpu.VMEM((2, CHUNK, C), jnp.float32),
            pltpu.VMEM((2, CHUNK, C), jnp.float32),
            pltpu.VMEM((CHUNK,), jnp.float32),
            pltpu.SemaphoreType.DMA((2,)),
        ],
    )(_sc_body)
    return kfn(a_flat, p_flat, sidx2d, fidx2d)


def _tc_post_body(lg_ref, typ_ref, bias_ref, logits_ref, loss_ref):
    lg = lg_ref[...]
    t = typ_ref[...]
    b4 = bias_ref[...]
    be = jnp.where(t == 0, b4[0:1],
                   jnp.where(t == 1, b4[1:2],
                             jnp.where(t == 2, b4[2:3], b4[3:4])))
    logit = lg + be
    logits_ref[...] = logit
    sp = jax.nn.softplus(-logit)
    loss_ref[...] = (jnp.sum(sp) / float(N_EDGES)).reshape(1, 1)


def _tc_post(lg2d, typ2d, bias_b):
    return pl.pallas_call(
        _tc_post_body,
        out_shape=[
            jax.ShapeDtypeStruct((ROWS_OUT, 128), jnp.float32),
            jax.ShapeDtypeStruct((1, 1), jnp.float32),
        ],
    )(lg2d, typ2d, bias_b)


def kernel(assignments, edge_index, edge_type, inter_cluster_logits,
           log_alpha, absent_bias):
    pad = E_PAD - N_EDGES
    srcp = jnp.pad(edge_index[0], (0, pad)).reshape(ROWS128, 128)
    dstp = jnp.pad(edge_index[1], (0, pad)).reshape(ROWS128, 128)
    typp = jnp.pad(edge_type, (0, pad)).reshape(ROWS128, 128)
    a_packed = assignments.reshape(NPACK, 128)

    z, l0, p_packed, fused = _tc_pre(inter_cluster_logits, log_alpha,
                                     a_packed, dstp, typp)
    lg_nb = _sc_main(a_packed.reshape(N_NODES, C),
                     p_packed.reshape(R * N_NODES, C), srcp, fused)

    lg2d = lg_nb.reshape(ROWS128, 128)[:ROWS_OUT]
    typ2d = edge_type.reshape(ROWS_OUT, 128)
    bias_b = jnp.broadcast_to(absent_bias[:, None], (R, 128))
    logits2d, loss11 = _tc_post(lg2d, typ2d, bias_b)
    return (loss11[0, 0], logits2d.reshape(N_EDGES), l0, z)


# speedup vs baseline: 1.0040x; 1.0040x over previous
import functools
import math

import jax
import jax.numpy as jnp
from jax import lax
from jax.experimental import pallas as pl
from jax.experimental.pallas import tpu as pltpu
from jax.experimental.pallas import tpu_sc as plsc

N_NODES = 50000
N_EDGES = 800000
R = 4
C = 8
TEMPERATURE = 2.0 / 3.0
LIMIT_A = -0.1
LIMIT_B = 1.1

E_PAD = 819200
N_WORKERS = 32
CHUNK = 1024
CHUNKS_PER_W = E_PAD // (N_WORKERS * CHUNK)
ROWS128 = E_PAD // 128
ROWS_OUT = N_EDGES // 128
NPACK = N_NODES // 16

_THRESHOLD = TEMPERATURE * math.log(-LIMIT_A / LIMIT_B)


def _tc_pre_body(icl_ref, la_ref, ap_ref, dst_ref, typ_ref,
                 z_ref, l0_ref, p_ref, fused_ref):
    @pl.when(pl.program_id(0) == 0)
    def _():
        la = la_ref[...]
        z = jnp.clip(jax.nn.sigmoid(la) * (LIMIT_B - LIMIT_A) + LIMIT_A,
                     0.0, 1.0)
        w = jax.nn.sigmoid(icl_ref[...]) * z
        z_ref[...] = z
        l0_ref[...] = jax.nn.sigmoid(la - _THRESHOLD)

        e1 = (lax.broadcasted_iota(jnp.int32, (128, C), 0) % C
              == lax.broadcasted_iota(jnp.int32, (128, C), 1)
              ).astype(jnp.float32)
        e2 = (lax.broadcasted_iota(jnp.int32, (C, 128), 0)
              == lax.broadcasted_iota(jnp.int32, (C, 128), 1) % C
              ).astype(jnp.float32)
        blk = (lax.broadcasted_iota(jnp.int32, (128, 128), 0) // C
               == lax.broadcasted_iota(jnp.int32, (128, 128), 1) // C
               ).astype(jnp.float32)

        ap = ap_ref[...]
        for t in range(R):
            wt_tiled = jnp.dot(jnp.dot(e1, w[t].T,
                                       preferred_element_type=jnp.float32),
                               e2, preferred_element_type=jnp.float32)
            bd = wt_tiled * blk
            p_ref[t] = jnp.dot(ap, bd, preferred_element_type=jnp.float32)

    fused_ref[...] = dst_ref[...] + N_NODES * typ_ref[...]


def _tc_pre(icl, la, a_packed, dst2d, typ2d):
    grid = 5
    be = ROWS128 // grid
    return pl.pallas_call(
        _tc_pre_body,
        grid=(grid,),
        in_specs=[
            pl.BlockSpec((R, C, C), lambda i: (0, 0, 0)),
            pl.BlockSpec((R, C, C), lambda i: (0, 0, 0)),
            pl.BlockSpec((NPACK, 128), lambda i: (0, 0)),
            pl.BlockSpec((be, 128), lambda i: (i, 0)),
            pl.BlockSpec((be, 128), lambda i: (i, 0)),
        ],
        out_specs=[
            pl.BlockSpec((R, C, C), lambda i: (0, 0, 0)),
            pl.BlockSpec((R, C, C), lambda i: (0, 0, 0)),
            pl.BlockSpec((R, NPACK, 128), lambda i: (0, 0, 0)),
            pl.BlockSpec((be, 128), lambda i: (i, 0)),
        ],
        out_shape=[
            jax.ShapeDtypeStruct((R, C, C), jnp.float32),
            jax.ShapeDtypeStruct((R, C, C), jnp.float32),
            jax.ShapeDtypeStruct((R, NPACK, 128), jnp.float32),
            jax.ShapeDtypeStruct((ROWS128, 128), jnp.int32),
        ],
    )(icl, la, a_packed, dst2d, typ2d)


_J = CHUNK // 128


def _sc_body(a_hbm, p_hbm, sidx_hbm, fidx_hbm, out_hbm,
             sidx_v, fidx_v, srows_v, drows_v, lg_v, sems):
    wid = lax.axis_index("s") * 2 + lax.axis_index("c")

    row_base = wid * (CHUNKS_PER_W * _J)
    pltpu.sync_copy(sidx_hbm.at[pl.ds(row_base, CHUNKS_PER_W * _J)], sidx_v)
    pltpu.sync_copy(fidx_hbm.at[pl.ds(row_base, CHUNKS_PER_W * _J)], fidx_v)

    def start_chunk(g, b):
        for j in range(_J):
            pltpu.async_copy(a_hbm.at[sidx_v.at[g * _J + j]],
                             srows_v.at[b].at[pl.ds(j * 128, 128)], sems.at[b])
            pltpu.async_copy(p_hbm.at[fidx_v.at[g * _J + j]],
                             drows_v.at[b].at[pl.ds(j * 128, 128)], sems.at[b])

    def finish_chunk(g, b):
        pltpu.make_async_copy(
            a_hbm.at[pl.ds(0, CHUNK)],
            srows_v.at[b].at[pl.ds(0, CHUNK)], sems.at[b]).wait()
        pltpu.make_async_copy(
            p_hbm.at[pl.ds(0, CHUNK)],
            drows_v.at[b].at[pl.ds(0, CHUNK)], sems.at[b]).wait()

        def vec_body(v, _):
            ids = lax.iota(jnp.int32, 16) + v * 16
            acc = jnp.zeros((16,), jnp.float32)
            for c in range(C):
                cc = jnp.full((16,), c, jnp.int32)
                s = plsc.load_gather(srows_v.at[b], [ids, cc])
                d = plsc.load_gather(drows_v.at[b], [ids, cc])
                acc = acc + s * d
            lg_v[pl.ds(v * 16, 16)] = acc
            return 0

        lax.fori_loop(0, CHUNK // 16, vec_body, 0, unroll=4)
        base = wid * (CHUNKS_PER_W * CHUNK) + g * CHUNK
        pltpu.sync_copy(lg_v, out_hbm.at[pl.ds(base, CHUNK)])

    start_chunk(0, 0)

    def pair_body(h, _):
        g = h * 2
        start_chunk(g + 1, 1)
        finish_chunk(g, 0)
        start_chunk(g + 2, 0)
        finish_chunk(g + 1, 1)
        return 0

    n_pairs = CHUNKS_PER_W // 2 - 1
    lax.fori_loop(0, n_pairs, pair_body, 0, unroll=False)
    g0 = 2 * n_pairs
    start_chunk(g0 + 1, 1)
    finish_chunk(g0, 0)
    if CHUNKS_PER_W % 2:
        start_chunk(g0 + 2, 0)
        finish_chunk(g0 + 1, 1)
        finish_chunk(g0 + 2, 0)
    else:
        finish_chunk(g0 + 1, 1)


def _sc_main(a_flat, p_flat, sidx2d, fidx2d):
    mesh = plsc.VectorSubcoreMesh(core_axis_name="c", subcore_axis_name="s")
    kfn = functools.partial(
        pl.kernel,
        out_type=jax.ShapeDtypeStruct((E_PAD,), jnp.float32),
        mesh=mesh,
        compiler_params=pltpu.CompilerParams(
            use_tc_tiling_on_sc=False, needs_layout_passes=False),
        scratch_types=[
            pltpu.VMEM((CHUNKS_PER_W * _J, 128), jnp.int32),
            pltpu.VMEM((CHUNKS_PER_W * _J, 128), jnp.int32),
            pltpu.VMEM((2, CHUNK, C), jnp.float32),
            pltpu.VMEM((2, CHUNK, C), jnp.float32),
            pltpu.VMEM((CHUNK,), jnp.float32),
            pltpu.SemaphoreType.DMA((2,)),
        ],
    )(_sc_body)
    return kfn(a_flat, p_flat, sidx2d, fidx2d)


def _tc_post_body(lg_ref, typ_ref, bias_ref, logits_ref, loss_ref):
    lg = lg_ref[...]
    t = typ_ref[...]
    b4 = bias_ref[...]
    be = jnp.where(t == 0, b4[0:1],
                   jnp.where(t == 1, b4[1:2],
                             jnp.where(t == 2, b4[2:3], b4[3:4])))
    logit = lg + be
    logits_ref[...] = logit
    sp = jax.nn.softplus(-logit)
    loss_ref[...] = (jnp.sum(sp) / float(N_EDGES)).reshape(1, 1)


def _tc_post(lg2d, typ2d, bias_b):
    return pl.pallas_call(
        _tc_post_body,
        out_shape=[
            jax.ShapeDtypeStruct((ROWS_OUT, 128), jnp.float32),
            jax.ShapeDtypeStruct((1, 1), jnp.float32),
        ],
    )(lg2d, typ2d, bias_b)


def kernel(assignments, edge_index, edge_type, inter_cluster_logits,
           log_alpha, absent_bias):
    pad = E_PAD - N_EDGES
    srcp = jnp.pad(edge_index[0], (0, pad)).reshape(ROWS128, 128)
    dstp = jnp.pad(edge_index[1], (0, pad)).reshape(ROWS128, 128)
    typp = jnp.pad(edge_type, (0, pad)).reshape(ROWS128, 128)
    a_packed = assignments.reshape(NPACK, 128)

    z, l0, p_packed, fused = _tc_pre(inter_cluster_logits, log_alpha,
                                     a_packed, dstp, typp)
    lg_nb = _sc_main(a_packed.reshape(N_NODES, C),
                     p_packed.reshape(R * N_NODES, C), srcp, fused)

    lg2d = lg_nb.reshape(ROWS128, 128)[:ROWS_OUT]
    typ2d = edge_type.reshape(ROWS_OUT, 128)
    bias_b = jnp.broadcast_to(absent_bias[:, None], (R, 128))
    logits2d, loss11 = _tc_post(lg2d, typ2d, bias_b)
    return (loss11[0, 0], logits2d.reshape(N_EDGES), l0, z)


# speedup vs baseline: 1.0401x; 1.0360x over previous
import functools
import math

import jax
import jax.numpy as jnp
from jax import lax
from jax.experimental import pallas as pl
from jax.experimental.pallas import tpu as pltpu
from jax.experimental.pallas import tpu_sc as plsc

N_NODES = 50000
N_EDGES = 800000
R = 4
C = 8
TEMPERATURE = 2.0 / 3.0
LIMIT_A = -0.1
LIMIT_B = 1.1

E_PAD = 819200
N_WORKERS = 32
CHUNK = 1024
CHUNKS_PER_W = E_PAD // (N_WORKERS * CHUNK)
ROWS128 = E_PAD // 128
ROWS_OUT = N_EDGES // 128
NPACK = N_NODES // 16

_THRESHOLD = TEMPERATURE * math.log(-LIMIT_A / LIMIT_B)


def _tc_pre_body(icl_ref, la_ref, ap_ref, dst_ref, typ_ref,
                 z_ref, l0_ref, p_ref, fused_ref):
    @pl.when(pl.program_id(0) == 0)
    def _():
        la = la_ref[...]
        z = jnp.clip(jax.nn.sigmoid(la) * (LIMIT_B - LIMIT_A) + LIMIT_A,
                     0.0, 1.0)
        w = jax.nn.sigmoid(icl_ref[...]) * z
        z_ref[...] = z
        l0_ref[...] = jax.nn.sigmoid(la - _THRESHOLD)

        e1 = (lax.broadcasted_iota(jnp.int32, (128, C), 0) % C
              == lax.broadcasted_iota(jnp.int32, (128, C), 1)
              ).astype(jnp.float32)
        e2 = (lax.broadcasted_iota(jnp.int32, (C, 128), 0)
              == lax.broadcasted_iota(jnp.int32, (C, 128), 1) % C
              ).astype(jnp.float32)
        blk = (lax.broadcasted_iota(jnp.int32, (128, 128), 0) // C
               == lax.broadcasted_iota(jnp.int32, (128, 128), 1) // C
               ).astype(jnp.float32)

        ap = ap_ref[...]
        for t in range(R):
            wt_tiled = jnp.dot(jnp.dot(e1, w[t].T,
                                       preferred_element_type=jnp.float32),
                               e2, preferred_element_type=jnp.float32)
            bd = wt_tiled * blk
            p_ref[t] = jnp.dot(ap, bd, preferred_element_type=jnp.float32)

    fused_ref[...] = dst_ref[...] + N_NODES * typ_ref[...]


def _tc_pre(icl, la, a_packed, dst2d, typ2d):
    grid = 5
    be = ROWS128 // grid
    return pl.pallas_call(
        _tc_pre_body,
        grid=(grid,),
        in_specs=[
            pl.BlockSpec((R, C, C), lambda i: (0, 0, 0)),
            pl.BlockSpec((R, C, C), lambda i: (0, 0, 0)),
            pl.BlockSpec((NPACK, 128), lambda i: (0, 0)),
            pl.BlockSpec((be, 128), lambda i: (i, 0)),
            pl.BlockSpec((be, 128), lambda i: (i, 0)),
        ],
        out_specs=[
            pl.BlockSpec((R, C, C), lambda i: (0, 0, 0)),
            pl.BlockSpec((R, C, C), lambda i: (0, 0, 0)),
            pl.BlockSpec((R, NPACK, 128), lambda i: (0, 0, 0)),
            pl.BlockSpec((be, 128), lambda i: (i, 0)),
        ],
        out_shape=[
            jax.ShapeDtypeStruct((R, C, C), jnp.float32),
            jax.ShapeDtypeStruct((R, C, C), jnp.float32),
            jax.ShapeDtypeStruct((R, NPACK, 128), jnp.float32),
            jax.ShapeDtypeStruct((ROWS128, 128), jnp.int32),
        ],
    )(icl, la, a_packed, dst2d, typ2d)


_J = CHUNK // 128


def _sc_body(a_hbm, p_hbm, sidx_hbm, fidx_hbm, out_hbm,
             sidx_v, fidx_v, srows_v, drows_v, lg_v, sems):
    wid = lax.axis_index("s") * 2 + lax.axis_index("c")

    row_base = wid * (CHUNKS_PER_W * _J)
    pltpu.sync_copy(sidx_hbm.at[pl.ds(row_base, CHUNKS_PER_W * _J)], sidx_v)
    pltpu.sync_copy(fidx_hbm.at[pl.ds(row_base, CHUNKS_PER_W * _J)], fidx_v)

    def start_chunk(g, b):
        for j in range(_J):
            pltpu.async_copy(a_hbm.at[sidx_v.at[g * _J + j]],
                             srows_v.at[b].at[pl.ds(j * 128, 128)], sems.at[b])
            pltpu.async_copy(p_hbm.at[fidx_v.at[g * _J + j]],
                             drows_v.at[b].at[pl.ds(j * 128, 128)], sems.at[b])

    def finish_chunk(g, b):
        pltpu.make_async_copy(
            a_hbm.at[pl.ds(0, CHUNK)],
            srows_v.at[b].at[pl.ds(0, CHUNK)], sems.at[b]).wait()
        pltpu.make_async_copy(
            p_hbm.at[pl.ds(0, CHUNK)],
            drows_v.at[b].at[pl.ds(0, CHUNK)], sems.at[b]).wait()

        def vec_body(v, _):
            ids = lax.iota(jnp.int32, 16) + v * 16
            acc = jnp.zeros((16,), jnp.float32)
            for c in range(C):
                cc = jnp.full((16,), c, jnp.int32)
                s = plsc.load_gather(srows_v.at[b], [ids, cc])
                d = plsc.load_gather(drows_v.at[b], [ids, cc])
                acc = acc + s * d
            lg_v[pl.ds(v * 16, 16)] = acc
            return 0

        lax.fori_loop(0, CHUNK // 16, vec_body, 0, unroll=8)
        base = wid * (CHUNKS_PER_W * CHUNK) + g * CHUNK
        pltpu.sync_copy(lg_v, out_hbm.at[pl.ds(base, CHUNK)])

    start_chunk(0, 0)

    def pair_body(h, _):
        g = h * 2
        start_chunk(g + 1, 1)
        finish_chunk(g, 0)
        start_chunk(g + 2, 0)
        finish_chunk(g + 1, 1)
        return 0

    n_pairs = CHUNKS_PER_W // 2 - 1
    lax.fori_loop(0, n_pairs, pair_body, 0, unroll=False)
    g0 = 2 * n_pairs
    start_chunk(g0 + 1, 1)
    finish_chunk(g0, 0)
    if CHUNKS_PER_W % 2:
        start_chunk(g0 + 2, 0)
        finish_chunk(g0 + 1, 1)
        finish_chunk(g0 + 2, 0)
    else:
        finish_chunk(g0 + 1, 1)


def _sc_main(a_flat, p_flat, sidx2d, fidx2d):
    mesh = plsc.VectorSubcoreMesh(core_axis_name="c", subcore_axis_name="s")
    kfn = functools.partial(
        pl.kernel,
        out_type=jax.ShapeDtypeStruct((E_PAD,), jnp.float32),
        mesh=mesh,
        compiler_params=pltpu.CompilerParams(
            use_tc_tiling_on_sc=False, needs_layout_passes=False),
        scratch_types=[
            pltpu.VMEM((CHUNKS_PER_W * _J, 128), jnp.int32),
            pltpu.VMEM((CHUNKS_PER_W * _J, 128), jnp.int32),
            pltpu.VMEM((2, CHUNK, C), jnp.float32),
            pltpu.VMEM((2, CHUNK, C), jnp.float32),
            pltpu.VMEM((CHUNK,), jnp.float32),
            pltpu.SemaphoreType.DMA((2,)),
        ],
    )(_sc_body)
    return kfn(a_flat, p_flat, sidx2d, fidx2d)


def _tc_post_body(lg_ref, typ_ref, bias_ref, logits_ref, loss_ref):
    lg = lg_ref[...]
    t = typ_ref[...]
    b4 = bias_ref[...]
    be = jnp.where(t == 0, b4[0:1],
                   jnp.where(t == 1, b4[1:2],
                             jnp.where(t == 2, b4[2:3], b4[3:4])))
    logit = lg + be
    logits_ref[...] = logit
    sp = jax.nn.softplus(-logit)
    loss_ref[...] = (jnp.sum(sp) / float(N_EDGES)).reshape(1, 1)


def _tc_post(lg2d, typ2d, bias_b):
    return pl.pallas_call(
        _tc_post_body,
        out_shape=[
            jax.ShapeDtypeStruct((ROWS_OUT, 128), jnp.float32),
            jax.ShapeDtypeStruct((1, 1), jnp.float32),
        ],
    )(lg2d, typ2d, bias_b)


def kernel(assignments, edge_index, edge_type, inter_cluster_logits,
           log_alpha, absent_bias):
    pad = E_PAD - N_EDGES
    srcp = jnp.pad(edge_index[0], (0, pad)).reshape(ROWS128, 128)
    dstp = jnp.pad(edge_index[1], (0, pad)).reshape(ROWS128, 128)
    typp = jnp.pad(edge_type, (0, pad)).reshape(ROWS128, 128)
    a_packed = assignments.reshape(NPACK, 128)

    z, l0, p_packed, fused = _tc_pre(inter_cluster_logits, log_alpha,
                                     a_packed, dstp, typp)
    lg_nb = _sc_main(a_packed.reshape(N_NODES, C),
                     p_packed.reshape(R * N_NODES, C), srcp, fused)

    lg2d = lg_nb.reshape(ROWS128, 128)[:ROWS_OUT]
    typ2d = edge_type.reshape(ROWS_OUT, 128)
    bias_b = jnp.broadcast_to(absent_bias[:, None], (R, 128))
    logits2d, loss11 = _tc_post(lg2d, typ2d, bias_b)
    return (loss11[0, 0], logits2d.reshape(N_EDGES), l0, z)


# speedup vs baseline: 1.3679x; 1.3151x over previous
import functools
import math

import jax
import jax.numpy as jnp
from jax import lax
from jax.experimental import pallas as pl
from jax.experimental.pallas import tpu as pltpu
from jax.experimental.pallas import tpu_sc as plsc

N_NODES = 50000
N_EDGES = 800000
R = 4
C = 8
TEMPERATURE = 2.0 / 3.0
LIMIT_A = -0.1
LIMIT_B = 1.1

E_PAD = 802816
N_WORKERS = 32
CHUNK = 896
CHUNKS_PER_W = E_PAD // (N_WORKERS * CHUNK)
ROWS128 = E_PAD // 128
ROWS_OUT = N_EDGES // 128
NPACK = N_NODES // 16

_THRESHOLD = TEMPERATURE * math.log(-LIMIT_A / LIMIT_B)


def _tc_pre_body(icl_ref, la_ref, ap_ref, dst_ref, typ_ref,
                 z_ref, l0_ref, p_ref, fused_ref):
    @pl.when(pl.program_id(0) == 0)
    def _():
        la = la_ref[...]
        z = jnp.clip(jax.nn.sigmoid(la) * (LIMIT_B - LIMIT_A) + LIMIT_A,
                     0.0, 1.0)
        w = jax.nn.sigmoid(icl_ref[...]) * z
        z_ref[...] = z
        l0_ref[...] = jax.nn.sigmoid(la - _THRESHOLD)

        e1 = (lax.broadcasted_iota(jnp.int32, (128, C), 0) % C
              == lax.broadcasted_iota(jnp.int32, (128, C), 1)
              ).astype(jnp.float32)
        e2 = (lax.broadcasted_iota(jnp.int32, (C, 128), 0)
              == lax.broadcasted_iota(jnp.int32, (C, 128), 1) % C
              ).astype(jnp.float32)
        blk = (lax.broadcasted_iota(jnp.int32, (128, 128), 0) // C
               == lax.broadcasted_iota(jnp.int32, (128, 128), 1) // C
               ).astype(jnp.float32)

        ap = ap_ref[...]
        for t in range(R):
            wt_tiled = jnp.dot(jnp.dot(e1, w[t].T,
                                       preferred_element_type=jnp.float32),
                               e2, preferred_element_type=jnp.float32)
            bd = wt_tiled * blk
            p_ref[t] = jnp.dot(ap, bd, preferred_element_type=jnp.float32)

    fused_ref[...] = dst_ref[...] + N_NODES * typ_ref[...]


def _tc_pre(icl, la, a_packed, dst2d, typ2d):
    grid = 7
    be = ROWS128 // grid
    return pl.pallas_call(
        _tc_pre_body,
        grid=(grid,),
        in_specs=[
            pl.BlockSpec((R, C, C), lambda i: (0, 0, 0)),
            pl.BlockSpec((R, C, C), lambda i: (0, 0, 0)),
            pl.BlockSpec((NPACK, 128), lambda i: (0, 0)),
            pl.BlockSpec((be, 128), lambda i: (i, 0)),
            pl.BlockSpec((be, 128), lambda i: (i, 0)),
        ],
        out_specs=[
            pl.BlockSpec((R, C, C), lambda i: (0, 0, 0)),
            pl.BlockSpec((R, C, C), lambda i: (0, 0, 0)),
            pl.BlockSpec((R, NPACK, 128), lambda i: (0, 0, 0)),
            pl.BlockSpec((be, 128), lambda i: (i, 0)),
        ],
        out_shape=[
            jax.ShapeDtypeStruct((R, C, C), jnp.float32),
            jax.ShapeDtypeStruct((R, C, C), jnp.float32),
            jax.ShapeDtypeStruct((R, NPACK, 128), jnp.float32),
            jax.ShapeDtypeStruct((ROWS128, 128), jnp.int32),
        ],
    )(icl, la, a_packed, dst2d, typ2d)


_J = CHUNK // 128


def _sc_body(a_hbm, p_hbm, sidx_hbm, fidx_hbm, out_hbm,
             sidx_v, fidx_v, srows_v, drows_v, lg_v, sems):
    wid = lax.axis_index("s") * 2 + lax.axis_index("c")

    row_base = wid * (CHUNKS_PER_W * _J)
    pltpu.sync_copy(sidx_hbm.at[pl.ds(row_base, CHUNKS_PER_W * _J)], sidx_v)
    pltpu.sync_copy(fidx_hbm.at[pl.ds(row_base, CHUNKS_PER_W * _J)], fidx_v)

    def start_chunk(g, b):
        for j in range(_J):
            pltpu.async_copy(a_hbm.at[sidx_v.at[g * _J + j]],
                             srows_v.at[b].at[pl.ds(j * 128, 128)], sems.at[b])
            pltpu.async_copy(p_hbm.at[fidx_v.at[g * _J + j]],
                             drows_v.at[b].at[pl.ds(j * 128, 128)], sems.at[b])

    def finish_chunk(g, b):
        pltpu.make_async_copy(
            a_hbm.at[pl.ds(0, CHUNK)],
            srows_v.at[b].at[pl.ds(0, CHUNK)], sems.at[b]).wait()
        pltpu.make_async_copy(
            p_hbm.at[pl.ds(0, CHUNK)],
            drows_v.at[b].at[pl.ds(0, CHUNK)], sems.at[b]).wait()

        def vec_body(v, _):
            ids = lax.iota(jnp.int32, 16) + v * 16
            acc = jnp.zeros((16,), jnp.float32)
            for c in range(C):
                cc = jnp.full((16,), c, jnp.int32)
                s = plsc.load_gather(srows_v.at[b], [ids, cc])
                d = plsc.load_gather(drows_v.at[b], [ids, cc])
                acc = acc + s * d
            lg_v[pl.ds(v * 16, 16)] = acc
            return 0

        lax.fori_loop(0, CHUNK // 16, vec_body, 0, unroll=8)
        base = wid * (CHUNKS_PER_W * CHUNK) + g * CHUNK
        pltpu.sync_copy(lg_v, out_hbm.at[pl.ds(base, CHUNK)])

    start_chunk(0, 0)

    def pair_body(h, _):
        g = h * 2
        start_chunk(g + 1, 1)
        finish_chunk(g, 0)
        start_chunk(g + 2, 0)
        finish_chunk(g + 1, 1)
        return 0

    n_pairs = CHUNKS_PER_W // 2 - 1
    lax.fori_loop(0, n_pairs, pair_body, 0, unroll=False)
    g0 = 2 * n_pairs
    start_chunk(g0 + 1, 1)
    finish_chunk(g0, 0)
    if CHUNKS_PER_W % 2:
        start_chunk(g0 + 2, 0)
        finish_chunk(g0 + 1, 1)
        finish_chunk(g0 + 2, 0)
    else:
        finish_chunk(g0 + 1, 1)


def _sc_main(a_flat, p_flat, sidx2d, fidx2d):
    mesh = plsc.VectorSubcoreMesh(core_axis_name="c", subcore_axis_name="s")
    kfn = functools.partial(
        pl.kernel,
        out_type=jax.ShapeDtypeStruct((E_PAD,), jnp.float32),
        mesh=mesh,
        compiler_params=pltpu.CompilerParams(
            use_tc_tiling_on_sc=False, needs_layout_passes=False),
        scratch_types=[
            pltpu.VMEM((CHUNKS_PER_W * _J, 128), jnp.int32),
            pltpu.VMEM((CHUNKS_PER_W * _J, 128), jnp.int32),
            pltpu.VMEM((2, CHUNK, C), jnp.float32),
            pltpu.VMEM((2, CHUNK, C), jnp.float32),
            pltpu.VMEM((CHUNK,), jnp.float32),
            pltpu.SemaphoreType.DMA((2,)),
        ],
    )(_sc_body)
    return kfn(a_flat, p_flat, sidx2d, fidx2d)


def _tc_post_body(lg_ref, typ_ref, bias_ref, logits_ref, loss_ref):
    lg = lg_ref[...]
    t = typ_ref[...]
    b4 = bias_ref[...]
    be = jnp.where(t == 0, b4[0:1],
                   jnp.where(t == 1, b4[1:2],
                             jnp.where(t == 2, b4[2:3], b4[3:4])))
    logit = lg + be
    logits_ref[...] = logit
    sp = jax.nn.softplus(-logit)
    loss_ref[...] = (jnp.sum(sp) / float(N_EDGES)).reshape(1, 1)


def _tc_post(lg2d, typ2d, bias_b):
    return pl.pallas_call(
        _tc_post_body,
        out_shape=[
            jax.ShapeDtypeStruct((ROWS_OUT, 128), jnp.float32),
            jax.ShapeDtypeStruct((1, 1), jnp.float32),
        ],
    )(lg2d, typ2d, bias_b)


def kernel(assignments, edge_index, edge_type, inter_cluster_logits,
           log_alpha, absent_bias):
    pad = E_PAD - N_EDGES
    srcp = jnp.pad(edge_index[0], (0, pad)).reshape(ROWS128, 128)
    dstp = jnp.pad(edge_index[1], (0, pad)).reshape(ROWS128, 128)
    typp = jnp.pad(edge_type, (0, pad)).reshape(ROWS128, 128)
    a_packed = assignments.reshape(NPACK, 128)

    z, l0, p_packed, fused = _tc_pre(inter_cluster_logits, log_alpha,
                                     a_packed, dstp, typp)
    lg_nb = _sc_main(a_packed.reshape(N_NODES, C),
                     p_packed.reshape(R * N_NODES, C), srcp, fused)

    lg2d = lg_nb.reshape(ROWS128, 128)[:ROWS_OUT]
    typ2d = edge_type.reshape(ROWS_OUT, 128)
    bias_b = jnp.broadcast_to(absent_bias[:, None], (R, 128))
    logits2d, loss11 = _tc_post(lg2d, typ2d, bias_b)
    return (loss11[0, 0], logits2d.reshape(N_EDGES), l0, z)
